# Initial kernel scaffold; baseline (speedup 1.0000x reference)
import jax
import jax.numpy as jnp
from jax.experimental import pallas as pl


def kernel(x, edge_index, edge_attr, batch, emb_table, W1, a_src1, a_dst1, b1, W2, a_src2, a_dst2, b2, W3, a_src3, a_dst3, b3, fcW, fcb):
    raise NotImplementedError("write your pallas kernel here")


# speedup vs baseline: 31.1750x; 31.1750x over previous
import functools
import math

import jax
import jax.numpy as jnp
from jax import lax
from jax.experimental import pallas as pl
from jax.experimental.pallas import tpu as pltpu
from jax.experimental.pallas import tpu_sc as plsc

N = 10000
NP = 10240
E = 320000
IN = 128
EMB = 64
HID = 128
G = 256
OUT = 64

NC = 2
NS = 16
NW = NC * NS
RW = NP // NW
EW = E // NW

_SQRT2_INV = 1.0 / math.sqrt(2.0)


def _gelu_exact(x):
    return 0.5 * x * (1.0 + lax.erf(x * _SQRT2_INV))


def _emb_proj_body(emb_ref, w_ref, as_ref, ad_ref, out_ref, s_ref, d_ref):
    t2 = jnp.dot(emb_ref[...], w_ref[...], preferred_element_type=jnp.float32)
    out_ref[...] = t2
    s_ref[...] = jnp.sum(t2 * as_ref[...], axis=1, keepdims=True)
    d_ref[...] = jnp.sum(t2 * ad_ref[...], axis=1, keepdims=True)


def _emb_proj(emb_table, w1b, asrc, adst):
    grid = 5
    blk = N // grid
    return pl.pallas_call(
        _emb_proj_body,
        grid=(grid,),
        in_specs=[
            pl.BlockSpec((blk, EMB), lambda i: (i, 0)),
            pl.BlockSpec((EMB, HID), lambda i: (0, 0)),
            pl.BlockSpec((1, HID), lambda i: (0, 0)),
            pl.BlockSpec((1, HID), lambda i: (0, 0)),
        ],
        out_specs=[
            pl.BlockSpec((blk, HID), lambda i: (i, 0)),
            pl.BlockSpec((blk, 1), lambda i: (i, 0)),
            pl.BlockSpec((blk, 1), lambda i: (i, 0)),
        ],
        out_shape=[
            jax.ShapeDtypeStruct((N, HID), jnp.float32),
            jax.ShapeDtypeStruct((N, 1), jnp.float32),
            jax.ShapeDtypeStruct((N, 1), jnp.float32),
        ],
    )(emb_table, w1b, asrc.reshape(1, HID), adst.reshape(1, HID))


def _feat_proj_body(x_ref, w_ref, as_ref, ad_ref, out_ref, s_ref, d_ref):
    x = x_ref[...]
    nrm = jnp.sqrt(jnp.sum(x * x, axis=1, keepdims=True))
    nrm = jnp.where(nrm == 0.0, 1e-08, nrm)
    xn = x / nrm
    fa = jnp.dot(xn, w_ref[...], preferred_element_type=jnp.float32)
    out_ref[...] = fa
    s_ref[...] = jnp.sum(fa * as_ref[...], axis=1, keepdims=True)
    d_ref[...] = jnp.sum(fa * ad_ref[...], axis=1, keepdims=True)


def _feat_proj(feat_p, w1a, asrc, adst):
    grid = 5
    blk = NP // grid
    return pl.pallas_call(
        _feat_proj_body,
        grid=(grid,),
        in_specs=[
            pl.BlockSpec((blk, IN), lambda i: (i, 0)),
            pl.BlockSpec((IN, HID), lambda i: (0, 0)),
            pl.BlockSpec((1, HID), lambda i: (0, 0)),
            pl.BlockSpec((1, HID), lambda i: (0, 0)),
        ],
        out_specs=[
            pl.BlockSpec((blk, HID), lambda i: (i, 0)),
            pl.BlockSpec((blk, 1), lambda i: (i, 0)),
            pl.BlockSpec((blk, 1), lambda i: (i, 0)),
        ],
        out_shape=[
            jax.ShapeDtypeStruct((NP, HID), jnp.float32),
            jax.ShapeDtypeStruct((NP, 1), jnp.float32),
            jax.ShapeDtypeStruct((NP, 1), jnp.float32),
        ],
    )(feat_p, w1a, asrc.reshape(1, HID), adst.reshape(1, HID))


def _fuse1_body(t2_hbm, fa_hbm, t2s_hbm, t2d_hbm, fas_hbm, fad_hbm, idx2_hbm,
                hw_hbm, as_hbm, ad_hbm,
                idxv, fbuf, t2sv, t2dv, asq, adq, sem):
    wid = lax.axis_index("s") * NC + lax.axis_index("c")
    r0 = wid * RW
    pltpu.sync_copy(idx2_hbm.at[wid], idxv)
    pltpu.sync_copy(fa_hbm.at[pl.ds(r0, RW)], fbuf)
    pltpu.sync_copy(t2s_hbm, t2sv)
    pltpu.sync_copy(t2d_hbm, t2dv)
    pltpu.sync_copy(fas_hbm.at[pl.ds(r0, RW)], asq)
    pltpu.sync_copy(fad_hbm.at[pl.ds(r0, RW)], adq)

    descs = [
        pltpu.async_copy(t2_hbm.at[idxv.at[j]],
                         fbuf.at[pl.ds(j * 64, 64)], sem, add=True)
        for j in range(5)
    ]

    for j in range(5):
        for k in range(4):
            iv = idxv[j, pl.ds(k * 16, 16)]
            sl = pl.ds(j * 64 + k * 16, 16)
            asq[sl] = asq[sl] + plsc.load_gather(t2sv, [iv])
            adq[sl] = adq[sl] + plsc.load_gather(t2dv, [iv])

    for d in descs:
        d.wait()
    pltpu.sync_copy(fbuf, hw_hbm.at[pl.ds(r0, RW)])
    pltpu.sync_copy(asq, as_hbm.at[pl.ds(r0, RW)])
    pltpu.sync_copy(adq, ad_hbm.at[pl.ds(r0, RW)])


def _fuse1(t2, fa, t2s, t2d, fas, fad, idx2):
    mesh = plsc.VectorSubcoreMesh(core_axis_name="c", subcore_axis_name="s")
    f = pl.kernel(
        _fuse1_body,
        out_type=(
            jax.ShapeDtypeStruct((NP, HID), jnp.float32),
            jax.ShapeDtypeStruct((NP,), jnp.float32),
            jax.ShapeDtypeStruct((NP,), jnp.float32),
        ),
        mesh=mesh,
        scratch_types=[
            pltpu.VMEM((5, 64), jnp.int32),
            pltpu.VMEM((RW, HID), jnp.float32),
            pltpu.VMEM((NP,), jnp.float32),
            pltpu.VMEM((NP,), jnp.float32),
            pltpu.VMEM((RW,), jnp.float32),
            pltpu.VMEM((RW,), jnp.float32),
            pltpu.SemaphoreType.DMA,
        ],
        compiler_params=pltpu.CompilerParams(needs_layout_passes=False),
    )
    return f(t2, fa, t2s, t2d, fas, fad, idx2)


CE = 2000
NB = 5


def _edge_body(hw_hbm, as_hbm, ad_hbm, src_hbm, dst_hbm,
               outp_hbm, denp_hbm,
               asv, adv, srcv, dstv, denv, exv, rowb, zb, gsem, ssem,
               shared):
    cid = lax.axis_index("c")
    sid = lax.axis_index("s")
    wid = sid * NC + cid

    def zrow(i, c):
        for k in range(8):
            zb[i, pl.ds(k * 16, 16)] = jnp.zeros((16,), jnp.float32)
        return c

    lax.fori_loop(0, 8, zrow, 0)

    def zshared(t, c):
        pltpu.sync_copy(zb, shared.at[pl.ds(sid * (NP // NS) + t * 8, 8)])
        return c

    lax.fori_loop(0, NP // NS // 8, zshared, 0)

    def zden(t, c):
        denv[pl.ds(t * 16, 16)] = jnp.zeros((16,), jnp.float32)
        return c

    lax.fori_loop(0, NP // 16, zden, 0)

    pltpu.sync_copy(as_hbm, asv)
    pltpu.sync_copy(ad_hbm, adv)

    plsc.subcore_barrier()

    for ch in range(EW // CE):
        e0 = wid * EW + ch * CE
        pltpu.sync_copy(src_hbm.at[pl.ds(e0, CE)], srcv)
        pltpu.sync_copy(dst_hbm.at[pl.ds(e0, CE)], dstv)

        def escal(i, c):
            sl = pl.ds(i * 16, 16)
            s = srcv[sl]
            d = dstv[sl]
            a1 = plsc.load_gather(asv, [s])
            a2 = plsc.load_gather(adv, [d])
            e = a1 + a2
            e = jnp.where(e >= 0.0, e, 0.2 * e)
            ex = jnp.exp(e)
            plsc.addupdate_scatter(denv, [d], ex)
            exv[sl] = ex
            return c

        lax.fori_loop(0, CE // 16, escal, 0)

        def outer(o, c):
            base = o * (16 * NB)
            descs = []
            for b in range(NB):
                s = srcv[pl.ds(base + b * 16, 16)]
                descs.append(pltpu.async_copy(hw_hbm.at[s], rowb.at[b], gsem))
            sdescs = []
            for b in range(NB):
                descs[b].wait()

                def srow(r, c2):
                    wi = jnp.full((16,), base + b * 16 + r, jnp.int32)
                    wb = plsc.load_gather(exv, [wi])
                    for k in range(8):
                        sl = pl.ds(k * 16, 16)
                        rowb[b, r, sl] = rowb[b, r, sl] * wb
                    return c2

                lax.fori_loop(0, 16, srow, 0)
                d = dstv[pl.ds(base + b * 16, 16)]
                sdescs.append(
                    pltpu.async_copy(rowb.at[b], shared.at[d], ssem, add=True))
            for b in range(NB):
                sdescs[b].wait()
            return c

        lax.fori_loop(0, CE // 16 // NB, outer, 0)

    plsc.subcore_barrier()

    stripe = NP // NS
    pltpu.sync_copy(shared.at[pl.ds(sid * stripe, stripe)],
                    outp_hbm.at[cid, pl.ds(sid * stripe, stripe)])
    pltpu.sync_copy(denv, denp_hbm.at[wid, 0])


def _edge_pass(hw, asq, adq, src, dst):
    mesh = plsc.VectorSubcoreMesh(core_axis_name="c", subcore_axis_name="s")
    f = pl.kernel(
        _edge_body,
        out_type=(
            jax.ShapeDtypeStruct((NC, NP, HID), jnp.float32),
            jax.ShapeDtypeStruct((NW, 1, NP), jnp.float32),
        ),
        mesh=mesh,
        scratch_types=[
            pltpu.VMEM((NP,), jnp.float32),
            pltpu.VMEM((NP,), jnp.float32),
            pltpu.VMEM((CE,), jnp.int32),
            pltpu.VMEM((CE,), jnp.int32),
            pltpu.VMEM((NP,), jnp.float32),
            pltpu.VMEM((CE,), jnp.float32),
            pltpu.VMEM((NB, 16, HID), jnp.float32),
            pltpu.VMEM((8, HID), jnp.float32),
            pltpu.SemaphoreType.DMA,
            pltpu.SemaphoreType.DMA,
            pltpu.VMEM_SHARED((NP, HID), jnp.float32),
        ],
        compiler_params=pltpu.CompilerParams(needs_layout_passes=False),
    )
    outp, denp3 = f(hw, asq, adq, src, dst)
    return outp, denp3.reshape(NW, NP)


def _combine_body(hw_ref, as_ref, ad_ref, outp_ref, denp_ref, b_ref,
                  wn_ref, an_src_ref, an_dst_ref,
                  hwn_ref, asn_ref, adn_ref):
    hw = hw_ref[...]
    t = as_ref[...] + ad_ref[...]
    exs = jnp.exp(jnp.where(t >= 0.0, t, 0.2 * t))
    num = outp_ref[0] + outp_ref[1] + exs * hw
    ones = jnp.ones((NW, 1), jnp.float32)
    den = lax.dot_general(denp_ref[...], ones, (((0,), (0,)), ((), ())),
                          preferred_element_type=jnp.float32)
    den = den + exs + 1e-16
    h = num / den + b_ref[...]
    hg = _gelu_exact(h)
    hn = jnp.dot(hg, wn_ref[...], preferred_element_type=jnp.float32)
    hwn_ref[...] = hn
    asn_ref[...] = jnp.sum(hn * an_src_ref[...], axis=1, keepdims=True)
    adn_ref[...] = jnp.sum(hn * an_dst_ref[...], axis=1, keepdims=True)


def _combine(hw, asq, adq, outp, denp, b, wn, an_src, an_dst):
    grid = 10
    blk = NP // grid
    outs = pl.pallas_call(
        _combine_body,
        grid=(grid,),
        in_specs=[
            pl.BlockSpec((blk, HID), lambda i: (i, 0)),
            pl.BlockSpec((blk, 1), lambda i: (i, 0)),
            pl.BlockSpec((blk, 1), lambda i: (i, 0)),
            pl.BlockSpec((NC, blk, HID), lambda i: (0, i, 0)),
            pl.BlockSpec((NW, blk), lambda i: (0, i)),
            pl.BlockSpec((1, HID), lambda i: (0, 0)),
            pl.BlockSpec((HID, HID), lambda i: (0, 0)),
            pl.BlockSpec((1, HID), lambda i: (0, 0)),
            pl.BlockSpec((1, HID), lambda i: (0, 0)),
        ],
        out_specs=[
            pl.BlockSpec((blk, HID), lambda i: (i, 0)),
            pl.BlockSpec((blk, 1), lambda i: (i, 0)),
            pl.BlockSpec((blk, 1), lambda i: (i, 0)),
        ],
        out_shape=[
            jax.ShapeDtypeStruct((NP, HID), jnp.float32),
            jax.ShapeDtypeStruct((NP, 1), jnp.float32),
            jax.ShapeDtypeStruct((NP, 1), jnp.float32),
        ],
    )(hw, asq.reshape(NP, 1), adq.reshape(NP, 1), outp, denp,
      b.reshape(1, HID), wn, an_src.reshape(1, HID), an_dst.reshape(1, HID))
    hwn, asn, adn = outs
    return hwn, asn.reshape(NP), adn.reshape(NP)


def _pool_body(hw_ref, as_ref, ad_ref, outp_ref, denp_ref, b_ref, batch_ref,
               pooled_ref):
    hw = hw_ref[...]
    t = as_ref[...] + ad_ref[...]
    exs = jnp.exp(jnp.where(t >= 0.0, t, 0.2 * t))
    num = outp_ref[0] + outp_ref[1] + exs * hw
    ones = jnp.ones((NW, 1), jnp.float32)
    den = lax.dot_general(denp_ref[...], ones, (((0,), (0,)), ((), ())),
                          preferred_element_type=jnp.float32)
    den = den + exs + 1e-16
    h = num / den + b_ref[...]
    hg = _gelu_exact(h)
    bt = batch_ref[...]
    gids = lax.broadcasted_iota(jnp.int32, (bt.shape[0], G), 1)
    oh = jnp.where(bt == gids, 1.0, 0.0).astype(jnp.float32)
    blkpool = lax.dot_general(oh, hg, (((0,), (0,)), ((), ())),
                              preferred_element_type=jnp.float32)
    i = pl.program_id(0)

    @pl.when(i == 0)
    def _():
        pooled_ref[...] = blkpool

    @pl.when(i != 0)
    def _():
        pooled_ref[...] = pooled_ref[...] + blkpool


def _pool(hw, asq, adq, outp, denp, b, batch_p):
    grid = 10
    blk = NP // grid
    return pl.pallas_call(
        _pool_body,
        grid=(grid,),
        in_specs=[
            pl.BlockSpec((blk, HID), lambda i: (i, 0)),
            pl.BlockSpec((blk, 1), lambda i: (i, 0)),
            pl.BlockSpec((blk, 1), lambda i: (i, 0)),
            pl.BlockSpec((NC, blk, HID), lambda i: (0, i, 0)),
            pl.BlockSpec((NW, blk), lambda i: (0, i)),
            pl.BlockSpec((1, HID), lambda i: (0, 0)),
            pl.BlockSpec((blk, 1), lambda i: (i, 0)),
        ],
        out_specs=pl.BlockSpec((G, HID), lambda i: (0, 0)),
        out_shape=jax.ShapeDtypeStruct((G, HID), jnp.float32),
    )(hw, asq.reshape(NP, 1), adq.reshape(NP, 1), outp, denp,
      b.reshape(1, HID), batch_p.reshape(NP, 1))


def _fc_body(p_ref, w_ref, b_ref, out_ref):
    o = jnp.dot(p_ref[...], w_ref[...], preferred_element_type=jnp.float32)
    o = o + b_ref[...]
    out_ref[...] = jnp.where(o >= 0.0, o, 0.01 * o)


def _fc(pooled, fcw, fcb):
    return pl.pallas_call(
        _fc_body,
        in_specs=[
            pl.BlockSpec((G, HID), lambda: (0, 0)),
            pl.BlockSpec((HID, OUT), lambda: (0, 0)),
            pl.BlockSpec((1, OUT), lambda: (0, 0)),
        ],
        out_specs=pl.BlockSpec((G, OUT), lambda: (0, 0)),
        out_shape=jax.ShapeDtypeStruct((G, OUT), jnp.float32),
    )(pooled, fcw, fcb.reshape(1, OUT))


@jax.jit
def kernel(x, edge_index, edge_attr, batch, emb_table,
           W1, a_src1, a_dst1, b1,
           W2, a_src2, a_dst2, b2,
           W3, a_src3, a_dst3, b3,
           fcW, fcb):
    idx = x[:, IN].astype(jnp.int32)
    feat = x[:, :IN]
    idx_p = jnp.concatenate([idx, jnp.arange(NP - N, dtype=jnp.int32)])
    idx2 = idx_p.reshape(NW, 5, 64)
    feat_p = jnp.pad(feat, ((0, NP - N), (0, 0)))
    src = edge_index[0].astype(jnp.int32)
    dst = edge_index[1].astype(jnp.int32)
    batch_p = jnp.concatenate(
        [batch.astype(jnp.int32), jnp.full((NP - N,), G, jnp.int32)])

    t2, t2s, t2d = _emb_proj(emb_table, W1[IN:], a_src1, a_dst1)
    fa, fas, fad = _feat_proj(feat_p, W1[:IN], a_src1, a_dst1)
    t2s_p = jnp.pad(t2s.reshape(N), (0, NP - N))
    t2d_p = jnp.pad(t2d.reshape(N), (0, NP - N))
    hw1, as1, ad1 = _fuse1(t2, fa, t2s_p, t2d_p,
                           fas.reshape(NP), fad.reshape(NP), idx2)

    outp1, denp1 = _edge_pass(hw1, as1, ad1, src, dst)
    hw2, as2, ad2 = _combine(hw1, as1, ad1, outp1, denp1, b1,
                             W2, a_src2, a_dst2)

    outp2, denp2 = _edge_pass(hw2, as2, ad2, src, dst)
    hw3, as3, ad3 = _combine(hw2, as2, ad2, outp2, denp2, b2,
                             W3, a_src3, a_dst3)

    outp3, denp3 = _edge_pass(hw3, as3, ad3, src, dst)
    pooled = _pool(hw3, as3, ad3, outp3, denp3, b3, batch_p)

    return _fc(pooled, fcW, fcb)
